# Initial kernel scaffold; baseline (speedup 1.0000x reference)
import jax
import jax.numpy as jnp
from jax.experimental import pallas as pl


def kernel(x, edge_index, edge_attr, W1, b1, W2, b2):
    raise NotImplementedError("write your pallas kernel here")


# speedup vs baseline: 2.3708x; 2.3708x over previous
import functools

import jax
import jax.numpy as jnp
from jax import lax
from jax.experimental import pallas as pl
from jax.experimental.pallas import tpu as pltpu
from jax.experimental.pallas import tpu_sc as plsc

N_NODES = 10000
N_EDGES = 320000
D = 128

_NC = 2
_NS = 16
_NW = _NC * _NS
_EPW = N_EDGES // _NW
_CH = 80
_NCHUNK = _EPW // _CH


def _proj_body(x_ref, wa_ref, wb_ref, p_ref, q_ref):
    xv = x_ref[...]
    p_ref[...] = jnp.dot(xv, wa_ref[...], preferred_element_type=jnp.float32)
    q_ref[...] = jnp.dot(xv, wb_ref[...], preferred_element_type=jnp.float32)


def _project_nodes(x, wa, wb):
    blk = 2000
    grid = N_NODES // blk
    return pl.pallas_call(
        _proj_body,
        grid=(grid,),
        in_specs=[
            pl.BlockSpec((blk, D), lambda i: (i, 0)),
            pl.BlockSpec((D, D), lambda i: (0, 0)),
            pl.BlockSpec((D, D), lambda i: (0, 0)),
        ],
        out_specs=[
            pl.BlockSpec((blk, D), lambda i: (i, 0)),
            pl.BlockSpec((blk, D), lambda i: (i, 0)),
        ],
        out_shape=[
            jax.ShapeDtypeStruct((N_NODES, D), jnp.float32),
            jax.ShapeDtypeStruct((N_NODES, D), jnp.float32),
        ],
    )(x, wa, wb)


def _gather_add_body(p_hbm, q_hbm, src_hbm, dst_hbm, out_hbm,
                     si_v, di_v, rs_v, rd_v, sem_s, sem_d):
    wid = lax.axis_index("s") * _NC + lax.axis_index("c")
    wbase = wid * _EPW

    def chunk_body(k, carry):
        base = wbase + k * _CH
        pltpu.sync_copy(src_hbm.at[pl.ds(base, _CH)], si_v)
        pltpu.sync_copy(dst_hbm.at[pl.ds(base, _CH)], di_v)
        cp_s = pltpu.async_copy(p_hbm.at[si_v], rs_v, sem_s)
        cp_d = pltpu.async_copy(q_hbm.at[di_v], rd_v, sem_d)
        cp_s.wait()
        cp_d.wait()

        def row_body(r, c2):
            for j in range(D // 16):
                sl = pl.ds(j * 16, 16)
                rs_v[r, sl] = rs_v[r, sl] + rd_v[r, sl]
            return c2

        lax.fori_loop(0, _CH, row_body, 0)
        pltpu.sync_copy(rs_v, out_hbm.at[pl.ds(base, _CH)])
        return carry

    lax.fori_loop(0, _NCHUNK, chunk_body, 0)


def _gather_add(p, q, src, dst):
    mesh = plsc.VectorSubcoreMesh(core_axis_name="c", subcore_axis_name="s")
    fn = functools.partial(
        pl.kernel,
        mesh=mesh,
        out_type=jax.ShapeDtypeStruct((N_EDGES, D), jnp.float32),
        scratch_types=[
            pltpu.VMEM((_CH,), jnp.int32),
            pltpu.VMEM((_CH,), jnp.int32),
            pltpu.VMEM((_CH, D), jnp.float32),
            pltpu.VMEM((_CH, D), jnp.float32),
            pltpu.SemaphoreType.DMA,
            pltpu.SemaphoreType.DMA,
        ],
    )(_gather_add_body)
    return fn(p, q, src, dst)


def _mlp_body(g_ref, ea_ref, w1c_ref, b1_ref, w2_ref, b2_ref, o_ref):
    h = g_ref[...] + jnp.dot(ea_ref[...], w1c_ref[...],
                             preferred_element_type=jnp.float32) + b1_ref[...]
    h = jnp.maximum(h, 0.0)
    o_ref[...] = jnp.dot(h, w2_ref[...],
                         preferred_element_type=jnp.float32) + b2_ref[...]


def _edge_mlp(g, ea, w1c, b1, w2, b2):
    blk = 2560
    grid = N_EDGES // blk
    de = ea.shape[1]
    eo = w2.shape[1]
    return pl.pallas_call(
        _mlp_body,
        grid=(grid,),
        in_specs=[
            pl.BlockSpec((blk, D), lambda i: (i, 0)),
            pl.BlockSpec((blk, de), lambda i: (i, 0)),
            pl.BlockSpec((de, D), lambda i: (0, 0)),
            pl.BlockSpec((1, D), lambda i: (0, 0)),
            pl.BlockSpec((D, eo), lambda i: (0, 0)),
            pl.BlockSpec((1, eo), lambda i: (0, 0)),
        ],
        out_specs=pl.BlockSpec((blk, eo), lambda i: (i, 0)),
        out_shape=jax.ShapeDtypeStruct((N_EDGES, eo), jnp.float32),
    )(g, ea, w1c, b1, w2, b2)


def kernel(x, edge_index, edge_attr, W1, b1, W2, b2):
    wa = W1[:D]
    wb = W1[D:2 * D]
    w1c = W1[2 * D:]
    p, q = _project_nodes(x, wa, wb)
    g = _gather_add(p, q, edge_index[0], edge_index[1])
    return _edge_mlp(g, edge_attr, w1c, b1.reshape(1, D), W2,
                     b2.reshape(1, -1))
